# baseline (speedup 1.0000x reference)
import functools
import math

import jax
import jax.numpy as jnp
from jax import lax
from jax.experimental import pallas as pl
from jax.experimental.pallas import tpu as pltpu
from jax.experimental.pallas import tpu_sc as plsc

NUM_ENT = 8000
N_NODES = 10000
HIDDEN = 200
NUM_TYPES = 16
E = 320000
BATCH = 1024
W = 128
DEG_COL = 72
RRELU_SLOPE = (1.0 / 8.0 + 1.0 / 3.0) / 2.0

E_PAD = 321536
N_SUPER = 157
CH_ROWS = 16
CORE0_N = 79
ACC_ROWS = 8008
ZS = 496
ZS_LAST = 568


def _gidx_body(src_ref, et_ref, dst_ref, out_ref, dstc_ref):
    out_ref[...] = et_ref[...] * N_NODES + src_ref[...]
    dstc_ref[...] = jnp.minimum(dst_ref[...], NUM_ENT)


def _build_gidx(src3, et3, dst3):
    return pl.pallas_call(
        _gidx_body,
        out_shape=[
            jax.ShapeDtypeStruct((N_SUPER, CH_ROWS, 128), jnp.int32),
            jax.ShapeDtypeStruct((N_SUPER, CH_ROWS, 128), jnp.int32),
        ],
    )(src3, et3, dst3)


_HT_BLK = 1000


def _ht_a_body(h_ref, wa_ref, outa_ref):
    outa_ref[...] = h_ref[...] * wa_ref[0]


def _ht_b_body(h_ref, wb_ref, outb_ref):
    hb = jnp.concatenate(
        [h_ref[...], jnp.ones((_HT_BLK, 1), jnp.float32),
         jnp.zeros((_HT_BLK, W - (HIDDEN - W) - 1), jnp.float32)], axis=1)
    outb_ref[...] = hb * wb_ref[0]


def _build_ht_one(body, h_part, w_part, width_in):
    nb = N_NODES // _HT_BLK
    return pl.pallas_call(
        body,
        grid=(nb, NUM_TYPES),
        in_specs=[
            pl.BlockSpec((_HT_BLK, width_in), lambda b, t: (b, 0)),
            pl.BlockSpec((1, 1, W), lambda b, t: (t, 0, 0)),
        ],
        out_specs=pl.BlockSpec((_HT_BLK, W), lambda b, t: (t * 10 + b, 0)),
        out_shape=jax.ShapeDtypeStruct((NUM_TYPES * N_NODES, W), jnp.float32),
    )(h_part, w_part.reshape(NUM_TYPES, 1, W))


def _sc_agg_body(ht_hbm, gidx_hbm, dst_hbm, z_hbm, out_a, out_b,
                 gbuf, dbuf, rows0, rows1, rows2, acc,
                 gsem0, gsem1, gsem2, ssem0, ssem1, ssem2):
    rows = [rows0, rows1, rows2]
    gsem = [gsem0, gsem1, gsem2]
    ssem = [ssem0, ssem1, ssem2]
    c = lax.axis_index("c")
    s = lax.axis_index("s")

    base0 = jnp.minimum(s, 15) * 5
    cnt0 = jnp.where(s < 15, 5, 4)
    base1 = CORE0_N + jnp.minimum(s, 14) * 5 + jnp.maximum(s - 14, 0) * 4
    cnt1 = jnp.where(s < 14, 5, 4)
    base = jnp.where(c == 0, base0, base1)
    cnt = jnp.where(c == 0, cnt0, cnt1)

    @pl.when(s < 15)
    def _():
        pltpu.sync_copy(z_hbm.at[pl.ds(0, ZS)], acc.at[pl.ds(s * ZS, ZS)])

    @pl.when(s == 15)
    def _():
        pltpu.sync_copy(z_hbm, acc.at[pl.ds(15 * ZS, ZS_LAST)])

    plsc.subcore_barrier()

    def body(i, carry):
        sck = base + i
        pltpu.sync_copy(gidx_hbm.at[sck], gbuf)
        pltpu.sync_copy(dst_hbm.at[sck], dbuf)
        for j0 in range(2):
            pltpu.async_copy(ht_hbm.at[gbuf.at[j0]], rows[j0], gsem[j0])
        for j in range(CH_ROWS):
            b = j % 3
            pltpu.make_async_copy(ht_hbm.at[gbuf.at[j]], rows[b],
                                  gsem[b]).wait()
            pltpu.async_copy(rows[b], acc.at[dbuf.at[j]], ssem[b], add=True)
            if j + 2 < CH_ROWS:
                nb = (j + 2) % 3
                if j >= 1:
                    pb = (j - 1) % 3
                    pltpu.make_async_copy(
                        rows[pb], acc.at[dbuf.at[j - 1]], ssem[pb]).wait()
                pltpu.async_copy(ht_hbm.at[gbuf.at[j + 2]], rows[nb], gsem[nb])
        for j in range(CH_ROWS - 3, CH_ROWS):
            b = j % 3
            pltpu.make_async_copy(rows[b], acc.at[dbuf.at[j]], ssem[b]).wait()
        return carry

    lax.fori_loop(0, cnt, body, 0)
    plsc.subcore_barrier()

    out = [out_a, out_b]
    for ci in range(2):
        @pl.when(jnp.logical_and(c == ci, s < 15))
        def _(ci=ci):
            sl = pl.ds(s * ZS, ZS)
            pltpu.sync_copy(acc.at[sl], out[ci].at[sl])

        @pl.when(jnp.logical_and(c == ci, s == 15))
        def _(ci=ci):
            sl = pl.ds(15 * ZS, ZS_LAST)
            pltpu.sync_copy(acc.at[sl], out[ci].at[sl])


def _sc_agg(ht, gidx3, dst3, zeros_slab):
    mesh = plsc.VectorSubcoreMesh(core_axis_name="c", subcore_axis_name="s")
    k = functools.partial(
        pl.kernel,
        out_type=tuple(
            jax.ShapeDtypeStruct((ACC_ROWS, W), jnp.float32)
            for _ in range(2)),
        mesh=mesh,
        scratch_types=[
            pltpu.VMEM((CH_ROWS, 128), jnp.int32),
            pltpu.VMEM((CH_ROWS, 128), jnp.int32),
            pltpu.VMEM((128, W), jnp.float32),
            pltpu.VMEM((128, W), jnp.float32),
            pltpu.VMEM((128, W), jnp.float32),
            pltpu.VMEM_SHARED((ACC_ROWS, W), jnp.float32),
        ] + [pltpu.SemaphoreType.DMA] * 6,
        compiler_params=pltpu.CompilerParams(use_tc_tiling_on_sc=True),
    )(_sc_agg_body)
    return k(ht, gidx3, dst3, zeros_slab)


def _decode_body(a0_ref, b0_ref, a1_ref, b1_ref, ei_ref, ri_ref, td_ref,
                 rel_ref, w1_ref, w2_ref, w3_ref, b_ref, temb_ref,
                 static_out, y_out):
    p0 = a0_ref[:NUM_ENT] + b0_ref[:NUM_ENT]
    p1 = a1_ref[:NUM_ENT] + b1_ref[:NUM_ENT]
    deg = p1[:, DEG_COL:DEG_COL + 1]
    norm = jnp.where(deg > 0, 1.0 / jnp.maximum(deg, 1.0), 0.0)
    st = jnp.concatenate([p0, p1[:, :HIDDEN - W]], axis=1) * norm
    st = jnp.where(st >= 0, st, st * RRELU_SLOPE)
    static_out[...] = st

    ei = ei_ref[...]
    acc = jnp.zeros((BATCH, HIDDEN), dtype=jnp.float32)
    chunk = 1000
    for k in range(NUM_ENT // chunk):
        iota = lax.broadcasted_iota(jnp.int32, (BATCH, chunk), 1) + k * chunk
        oh = (ei == iota).astype(jnp.float32)
        acc = acc + jnp.dot(oh, st[k * chunk:(k + 1) * chunk, :],
                            preferred_element_type=jnp.float32)
    ent = jnp.tanh(acc)

    ri = ri_ref[...]
    iota_r = lax.broadcasted_iota(jnp.int32, (BATCH, 230), 1)
    oh_r = (ri == iota_r).astype(jnp.float32)
    rel = jnp.dot(oh_r, rel_ref[...], preferred_element_type=jnp.float32)

    ti = td_ref[...] // 24
    iota_t = lax.broadcasted_iota(jnp.int32, (BATCH, 365), 1)
    oh_t = (ti == iota_t).astype(jnp.float32)
    tim = jnp.dot(oh_t, temb_ref[...], preferred_element_type=jnp.float32)

    x = (jnp.dot(ent, w1_ref[...], preferred_element_type=jnp.float32)
         + jnp.dot(rel, w2_ref[...], preferred_element_type=jnp.float32)
         + tim * w3_ref[...] + b_ref[...])
    y_out[...] = jnp.maximum(x, 0.0)


def _decode(a0, b0, a1, b1, ei, ri, td, rel_emb, w1, w2, w3, b, temb):
    return pl.pallas_call(
        _decode_body,
        out_shape=[
            jax.ShapeDtypeStruct((NUM_ENT, HIDDEN), jnp.float32),
            jax.ShapeDtypeStruct((BATCH, HIDDEN), jnp.float32),
        ],
    )(a0, b0, a1, b1, ei, ri, td, rel_emb, w1, w2, w3, b, temb)


def _score_body(y_ref, s_ref, out_ref):
    out_ref[...] = lax.dot_general(
        y_ref[...], s_ref[...], (((1,), (1,)), ((), ())),
        preferred_element_type=jnp.float32)


def _score(y, static):
    return pl.pallas_call(
        _score_body,
        out_shape=jax.ShapeDtypeStruct((BATCH, NUM_ENT), jnp.float32),
        compiler_params=pltpu.CompilerParams(
            vmem_limit_bytes=100 * 1024 * 1024),
    )(y, static)


def kernel(ent_emb, attr_emb, rel_emb, rgcn_weight, dec_W, dec_b, time_emb,
           edge_index, edge_type, batch_data):
    f32 = jnp.float32
    h = jnp.concatenate([ent_emb, attr_emb], axis=0)
    w = rgcn_weight.reshape(NUM_TYPES, HIDDEN)

    w_a = w[:, :W]
    w_b = jnp.concatenate(
        [w[:, W:], jnp.ones((NUM_TYPES, 1), f32),
         jnp.zeros((NUM_TYPES, W - (HIDDEN - W) - 1), f32)], axis=1)

    pad = E_PAD - E
    src3 = jnp.pad(edge_index[0], (0, pad)).reshape(N_SUPER, CH_ROWS, 128)
    et3 = jnp.pad(edge_type, (0, pad)).reshape(N_SUPER, CH_ROWS, 128)
    dst3 = jnp.pad(edge_index[1], (0, pad),
                   constant_values=NUM_ENT).reshape(N_SUPER, CH_ROWS, 128)

    ht_a = _build_ht_one(_ht_a_body, h[:, :W], w_a, W)
    gidx3, dstc3 = _build_gidx(src3, et3, dst3)
    zslab = jnp.zeros((ZS_LAST, W), f32)
    pa0, pb0 = _sc_agg(ht_a, gidx3, dstc3, zslab)
    ht_b = _build_ht_one(_ht_b_body, h[:, W:], w_b, HIDDEN - W)
    pa1, pb1 = _sc_agg(ht_b, gidx3, dstc3, zslab)

    ei = batch_data[:, 0:1]
    ri = batch_data[:, 1:2]
    td = batch_data[:, 3:4]
    w1 = dec_W[0:HIDDEN]
    w2 = dec_W[HIDDEN:2 * HIDDEN]
    w3 = dec_W[2 * HIDDEN:2 * HIDDEN + 1]
    b = dec_b.reshape(1, HIDDEN)

    static, y = _decode(pa0, pb0, pa1, pb1,
                        ei, ri, td, rel_emb, w1, w2, w3, b, time_emb)
    return _score(y, static)


# speedup vs baseline: 1.2315x; 1.2315x over previous
import functools
import math

import jax
import jax.numpy as jnp
from jax import lax
from jax.experimental import pallas as pl
from jax.experimental.pallas import tpu as pltpu
from jax.experimental.pallas import tpu_sc as plsc

NUM_ENT = 8000
N_NODES = 10000
HIDDEN = 200
NUM_TYPES = 16
E = 320000
BATCH = 1024
W = 128
DEG_COL = 72
RRELU_SLOPE = (1.0 / 8.0 + 1.0 / 3.0) / 2.0

N_SUPER = 250
CH_ROWS = 10
PER_CORE = 125
ACC_ROWS = 8008
ZS = 496
ZS_LAST = 568


def _gidx_body(src_ref, et_ref, dst_ref, out_ref, dstc_ref):
    out_ref[...] = et_ref[...] * N_NODES + src_ref[...]
    dstc_ref[...] = jnp.minimum(dst_ref[...], NUM_ENT)


def _build_gidx(src3, et3, dst3):
    return pl.pallas_call(
        _gidx_body,
        out_shape=[
            jax.ShapeDtypeStruct((N_SUPER, CH_ROWS, 128), jnp.int32),
            jax.ShapeDtypeStruct((N_SUPER, CH_ROWS, 128), jnp.int32),
        ],
    )(src3, et3, dst3)


_HT_BLK = 1000


def _ht_a_body(h_ref, wa_ref, outa_ref):
    outa_ref[...] = h_ref[...] * wa_ref[0]


def _ht_b_body(h_ref, wb_ref, outb_ref):
    hb = jnp.concatenate(
        [h_ref[...], jnp.ones((_HT_BLK, 1), jnp.float32),
         jnp.zeros((_HT_BLK, W - (HIDDEN - W) - 1), jnp.float32)], axis=1)
    outb_ref[...] = hb * wb_ref[0]


def _build_ht_one(body, h_part, w_part, width_in):
    nb = N_NODES // _HT_BLK
    return pl.pallas_call(
        body,
        grid=(nb, NUM_TYPES),
        in_specs=[
            pl.BlockSpec((_HT_BLK, width_in), lambda b, t: (b, 0)),
            pl.BlockSpec((1, 1, W), lambda b, t: (t, 0, 0)),
        ],
        out_specs=pl.BlockSpec((_HT_BLK, W), lambda b, t: (t * 10 + b, 0)),
        out_shape=jax.ShapeDtypeStruct((NUM_TYPES * N_NODES, W), jnp.float32),
    )(h_part, w_part.reshape(NUM_TYPES, 1, W))


def _sc_agg_body(ht_hbm, gidx_hbm, dst_hbm, z_hbm, out_a, out_b,
                 gbuf, dbuf, rows0, rows1, rows2, acc,
                 gsem0, gsem1, gsem2, ssem0, ssem1, ssem2):
    rows = [rows0, rows1, rows2]
    gsem = [gsem0, gsem1, gsem2]
    ssem = [ssem0, ssem1, ssem2]
    c = lax.axis_index("c")
    s = lax.axis_index("s")

    base = c * PER_CORE + jnp.minimum(s, 13) * 8 + jnp.maximum(s - 13, 0) * 7
    cnt = jnp.where(s < 13, 8, 7)

    @pl.when(s < 15)
    def _():
        pltpu.sync_copy(z_hbm.at[pl.ds(0, ZS)], acc.at[pl.ds(s * ZS, ZS)])

    @pl.when(s == 15)
    def _():
        pltpu.sync_copy(z_hbm, acc.at[pl.ds(15 * ZS, ZS_LAST)])

    plsc.subcore_barrier()

    def body(i, carry):
        sck = base + i
        pltpu.sync_copy(gidx_hbm.at[sck], gbuf)
        pltpu.sync_copy(dst_hbm.at[sck], dbuf)
        for j0 in range(2):
            pltpu.async_copy(ht_hbm.at[gbuf.at[j0]], rows[j0], gsem[j0])
        for j in range(CH_ROWS):
            b = j % 3
            pltpu.make_async_copy(ht_hbm.at[gbuf.at[j]], rows[b],
                                  gsem[b]).wait()
            pltpu.async_copy(rows[b], acc.at[dbuf.at[j]], ssem[b], add=True)
            if j + 2 < CH_ROWS:
                nb = (j + 2) % 3
                if j >= 1:
                    pb = (j - 1) % 3
                    pltpu.make_async_copy(
                        rows[pb], acc.at[dbuf.at[j - 1]], ssem[pb]).wait()
                pltpu.async_copy(ht_hbm.at[gbuf.at[j + 2]], rows[nb], gsem[nb])
        for j in range(CH_ROWS - 3, CH_ROWS):
            b = j % 3
            pltpu.make_async_copy(rows[b], acc.at[dbuf.at[j]], ssem[b]).wait()
        return carry

    lax.fori_loop(0, cnt, body, 0)
    plsc.subcore_barrier()

    out = [out_a, out_b]
    for ci in range(2):
        @pl.when(jnp.logical_and(c == ci, s < 15))
        def _(ci=ci):
            sl = pl.ds(s * ZS, ZS)
            pltpu.sync_copy(acc.at[sl], out[ci].at[sl])

        @pl.when(jnp.logical_and(c == ci, s == 15))
        def _(ci=ci):
            sl = pl.ds(15 * ZS, ZS_LAST)
            pltpu.sync_copy(acc.at[sl], out[ci].at[sl])


def _sc_agg(ht, gidx3, dst3, zeros_slab):
    mesh = plsc.VectorSubcoreMesh(core_axis_name="c", subcore_axis_name="s")
    k = functools.partial(
        pl.kernel,
        out_type=tuple(
            jax.ShapeDtypeStruct((ACC_ROWS, W), jnp.float32)
            for _ in range(2)),
        mesh=mesh,
        scratch_types=[
            pltpu.VMEM((CH_ROWS, 128), jnp.int32),
            pltpu.VMEM((CH_ROWS, 128), jnp.int32),
            pltpu.VMEM((128, W), jnp.float32),
            pltpu.VMEM((128, W), jnp.float32),
            pltpu.VMEM((128, W), jnp.float32),
            pltpu.VMEM_SHARED((ACC_ROWS, W), jnp.float32),
        ] + [pltpu.SemaphoreType.DMA] * 6,
        compiler_params=pltpu.CompilerParams(use_tc_tiling_on_sc=True),
    )(_sc_agg_body)
    return k(ht, gidx3, dst3, zeros_slab)


def _decode_body(a0_ref, b0_ref, a1_ref, b1_ref, ei_ref, ri_ref, td_ref,
                 rel_ref, w1_ref, w2_ref, w3_ref, b_ref, temb_ref,
                 static_out, y_out):
    p0 = a0_ref[:NUM_ENT] + b0_ref[:NUM_ENT]
    p1 = a1_ref[:NUM_ENT] + b1_ref[:NUM_ENT]
    deg = p1[:, DEG_COL:DEG_COL + 1]
    norm = jnp.where(deg > 0, 1.0 / jnp.maximum(deg, 1.0), 0.0)
    st = jnp.concatenate([p0, p1[:, :HIDDEN - W]], axis=1) * norm
    st = jnp.where(st >= 0, st, st * RRELU_SLOPE)
    static_out[...] = st

    ei = ei_ref[...]
    acc = jnp.zeros((BATCH, HIDDEN), dtype=jnp.float32)
    chunk = 1000
    for k in range(NUM_ENT // chunk):
        iota = lax.broadcasted_iota(jnp.int32, (BATCH, chunk), 1) + k * chunk
        oh = (ei == iota).astype(jnp.float32)
        acc = acc + jnp.dot(oh, st[k * chunk:(k + 1) * chunk, :],
                            preferred_element_type=jnp.float32)
    ent = jnp.tanh(acc)

    ri = ri_ref[...]
    iota_r = lax.broadcasted_iota(jnp.int32, (BATCH, 230), 1)
    oh_r = (ri == iota_r).astype(jnp.float32)
    rel = jnp.dot(oh_r, rel_ref[...], preferred_element_type=jnp.float32)

    ti = td_ref[...] // 24
    iota_t = lax.broadcasted_iota(jnp.int32, (BATCH, 365), 1)
    oh_t = (ti == iota_t).astype(jnp.float32)
    tim = jnp.dot(oh_t, temb_ref[...], preferred_element_type=jnp.float32)

    x = (jnp.dot(ent, w1_ref[...], preferred_element_type=jnp.float32)
         + jnp.dot(rel, w2_ref[...], preferred_element_type=jnp.float32)
         + tim * w3_ref[...] + b_ref[...])
    y_out[...] = jnp.maximum(x, 0.0)


def _decode(a0, b0, a1, b1, ei, ri, td, rel_emb, w1, w2, w3, b, temb):
    return pl.pallas_call(
        _decode_body,
        out_shape=[
            jax.ShapeDtypeStruct((NUM_ENT, HIDDEN), jnp.float32),
            jax.ShapeDtypeStruct((BATCH, HIDDEN), jnp.float32),
        ],
    )(a0, b0, a1, b1, ei, ri, td, rel_emb, w1, w2, w3, b, temb)


def _score_body(y_ref, s_ref, out_ref):
    out_ref[...] = lax.dot_general(
        y_ref[...], s_ref[...], (((1,), (1,)), ((), ())),
        preferred_element_type=jnp.float32)


def _score(y, static):
    return pl.pallas_call(
        _score_body,
        out_shape=jax.ShapeDtypeStruct((BATCH, NUM_ENT), jnp.float32),
        compiler_params=pltpu.CompilerParams(
            vmem_limit_bytes=100 * 1024 * 1024),
    )(y, static)


def kernel(ent_emb, attr_emb, rel_emb, rgcn_weight, dec_W, dec_b, time_emb,
           edge_index, edge_type, batch_data):
    f32 = jnp.float32
    h = jnp.concatenate([ent_emb, attr_emb], axis=0)
    w = rgcn_weight.reshape(NUM_TYPES, HIDDEN)

    w_a = w[:, :W]
    w_b = jnp.concatenate(
        [w[:, W:], jnp.ones((NUM_TYPES, 1), f32),
         jnp.zeros((NUM_TYPES, W - (HIDDEN - W) - 1), f32)], axis=1)

    src3 = edge_index[0].reshape(N_SUPER, CH_ROWS, 128)
    et3 = edge_type.reshape(N_SUPER, CH_ROWS, 128)
    dst3 = edge_index[1].reshape(N_SUPER, CH_ROWS, 128)

    gidx3, dstc3 = _build_gidx(src3, et3, dst3)
    zslab = jnp.zeros((ZS_LAST, W), f32)
    ht_a = _build_ht_one(_ht_a_body, h[:, :W], w_a, W)
    pa0, pb0 = _sc_agg(ht_a, gidx3, dstc3, zslab)
    ht_b = _build_ht_one(_ht_b_body, h[:, W:], w_b, HIDDEN - W)
    pa1, pb1 = _sc_agg(ht_b, gidx3, dstc3, zslab)

    ei = batch_data[:, 0:1]
    ri = batch_data[:, 1:2]
    td = batch_data[:, 3:4]
    w1 = dec_W[0:HIDDEN]
    w2 = dec_W[HIDDEN:2 * HIDDEN]
    w3 = dec_W[2 * HIDDEN:2 * HIDDEN + 1]
    b = dec_b.reshape(1, HIDDEN)

    static, y = _decode(pa0, pb0, pa1, pb1,
                        ei, ri, td, rel_emb, w1, w2, w3, b, time_emb)
    return _score(y, static)
